# Initial kernel scaffold; baseline (speedup 1.0000x reference)
import jax
import jax.numpy as jnp
from jax.experimental import pallas as pl


def kernel(x, edge_index, batch, W0, b0, W1a, b1a, W1b, b1b, W2a, b2a, W2b, b2b, Wh, bh, Wo, bo):
    raise NotImplementedError("write your pallas kernel here")


# speedup vs baseline: 9.6023x; 9.6023x over previous
import functools

import jax
import jax.numpy as jnp
from jax import lax
from jax.experimental import pallas as pl
from jax.experimental.pallas import tpu as pltpu
from jax.experimental.pallas import tpu_sc as plsc

_N = 10000
_E = 320000
_D = 128
_G = 64

_NC = 2
_NS = 16
_NW = _NC * _NS
_LN = 128
_CHUNKS = 79
_EPT = _CHUNKS * _LN
_EPAD = _NW * _EPT
_NPAD = 10008
_NACC = 10112
_RPT = _NACC // _NS


def _mesh():
    return plsc.VectorSubcoreMesh(core_axis_name="c", subcore_axis_name="s",
                                  num_cores=_NC, num_subcores=_NS)


@functools.cache
def _sc_degree_kernel():
    return functools.partial(
        pl.kernel,
        out_type=jax.ShapeDtypeStruct((_NC, _NACC), jnp.float32),
        mesh=_mesh(),
        scratch_types=[
            pltpu.VMEM((_CHUNKS, _LN), jnp.int32),
            pltpu.VMEM((_LN,), jnp.float32),
            pltpu.VMEM_SHARED((_NACC,), jnp.float32),
        ],
    )(_sc_degree_body)


def _sc_degree_body(dst_hbm, zeros1_hbm, out_hbm, didx_v, ones_v, dacc_sh):
    c = lax.axis_index("c")
    s = lax.axis_index("s")
    wid = s * _NC + c
    for k in range(_LN // 16):
        ones_v[pl.ds(16 * k, 16)] = jnp.full((16,), 1.0, jnp.float32)
    @pl.when(s == 0)
    def _():
        pltpu.sync_copy(zeros1_hbm, dacc_sh)
    plsc.subcore_barrier()
    pltpu.sync_copy(dst_hbm.at[wid], didx_v)

    def body(j, _):
        pltpu.sync_copy(ones_v, dacc_sh.at[didx_v.at[j]], add=True)
        return 0

    lax.fori_loop(0, _CHUNKS, body, 0)
    plsc.subcore_barrier()
    @pl.when(s == 0)
    def _():
        pltpu.sync_copy(dacc_sh, out_hbm.at[c])


@functools.cache
def _sc_edge_scatter_kernel():
    return functools.partial(
        pl.kernel,
        out_type=jax.ShapeDtypeStruct((_NC, _NACC, _D), jnp.float32),
        mesh=_mesh(),
        scratch_types=[
            pltpu.VMEM((_CHUNKS, _LN), jnp.int32),
            pltpu.VMEM((_CHUNKS, _LN), jnp.int32),
            pltpu.VMEM((_LN, _D), jnp.float32),
            pltpu.VMEM_SHARED((_NACC, _D), jnp.float32),
            pltpu.SemaphoreType.DMA,
        ],
    )(_sc_edge_scatter_body)


def _sc_edge_scatter_body(table_hbm, src_hbm, dst_hbm, zeros2_hbm, out_hbm,
                          sidx_v, didx_v, rows_v, acc_sh, sem):
    c = lax.axis_index("c")
    s = lax.axis_index("s")
    wid = s * _NC + c
    @pl.when(s == 0)
    def _():
        pltpu.sync_copy(zeros2_hbm, acc_sh)
    pltpu.sync_copy(src_hbm.at[wid], sidx_v)
    pltpu.sync_copy(dst_hbm.at[wid], didx_v)
    plsc.subcore_barrier()

    def body(j, _):
        pltpu.async_copy(table_hbm.at[sidx_v.at[j]], rows_v, sem).wait()
        pltpu.sync_copy(rows_v, acc_sh.at[didx_v.at[j]], add=True)
        return 0

    lax.fori_loop(0, _CHUNKS, body, 0)
    plsc.subcore_barrier()
    base = s * _RPT
    pltpu.sync_copy(acc_sh.at[pl.ds(base, _RPT)],
                    out_hbm.at[c].at[pl.ds(base, _RPT)])


_BLK = 1000


def _tc_gcn_pre_body(x_ref, invb_ref, w_ref, b_ref, hs_ref, self_ref):
    h = jnp.dot(x_ref[...], w_ref[...], preferred_element_type=jnp.float32)
    inv = invb_ref[...]
    hs_ref[...] = inv * h
    self_ref[...] = inv * inv * h + b_ref[...]


def _tc_gcn_pre(x, invb, w0, b0):
    return pl.pallas_call(
        _tc_gcn_pre_body,
        grid=(_N // _BLK,),
        in_specs=[
            pl.BlockSpec((_BLK, _D), lambda i: (i, 0)),
            pl.BlockSpec((_BLK, _D), lambda i: (i, 0)),
            pl.BlockSpec((_D, _D), lambda i: (0, 0)),
            pl.BlockSpec((1, _D), lambda i: (0, 0)),
        ],
        out_specs=[
            pl.BlockSpec((_BLK, _D), lambda i: (i, 0)),
            pl.BlockSpec((_BLK, _D), lambda i: (i, 0)),
        ],
        out_shape=[
            jax.ShapeDtypeStruct((_NPAD, _D), jnp.float32),
            jax.ShapeDtypeStruct((_N, _D), jnp.float32),
        ],
    )(x, invb, w0, b0)


def _tc_gcn_post_body(acc_ref, self_ref, invb_ref, h1_ref):
    agg = acc_ref[0] + acc_ref[1]
    h1_ref[...] = jnp.maximum(invb_ref[...] * agg + self_ref[...], 0.0)


def _tc_gcn_post(acc, selfterm, invb):
    return pl.pallas_call(
        _tc_gcn_post_body,
        grid=(_N // _BLK,),
        in_specs=[
            pl.BlockSpec((_NC, _BLK, _D), lambda i: (0, i, 0)),
            pl.BlockSpec((_BLK, _D), lambda i: (i, 0)),
            pl.BlockSpec((_BLK, _D), lambda i: (i, 0)),
        ],
        out_specs=pl.BlockSpec((_BLK, _D), lambda i: (i, 0)),
        out_shape=jax.ShapeDtypeStruct((_NPAD, _D), jnp.float32),
    )(acc, selfterm, invb)


def _tc_gin_body(h_ref, acc_ref, wa_ref, ba_ref, wb_ref, bb_ref, out_ref):
    t = h_ref[...] + acc_ref[0] + acc_ref[1]
    u = jnp.maximum(
        jnp.dot(t, wa_ref[...], preferred_element_type=jnp.float32)
        + ba_ref[...], 0.0)
    out_ref[...] = jnp.maximum(
        jnp.dot(u, wb_ref[...], preferred_element_type=jnp.float32)
        + bb_ref[...], 0.0)


def _tc_gin(h, acc, wa, ba, wb, bb):
    return pl.pallas_call(
        _tc_gin_body,
        grid=(_N // _BLK,),
        in_specs=[
            pl.BlockSpec((_BLK, _D), lambda i: (i, 0)),
            pl.BlockSpec((_NC, _BLK, _D), lambda i: (0, i, 0)),
            pl.BlockSpec((_D, _D), lambda i: (0, 0)),
            pl.BlockSpec((1, _D), lambda i: (0, 0)),
            pl.BlockSpec((_D, _D), lambda i: (0, 0)),
            pl.BlockSpec((1, _D), lambda i: (0, 0)),
        ],
        out_specs=pl.BlockSpec((_BLK, _D), lambda i: (i, 0)),
        out_shape=jax.ShapeDtypeStruct((_NPAD, _D), jnp.float32),
    )(h, acc, wa, ba, wb, bb)


def _tc_final_body(h_ref, acc_ref, wa_ref, ba_ref, wb_ref, bb_ref,
                   batch_ref, wh_ref, bh_ref, wo_ref, bo_ref, out_ref,
                   sums_ref, cnt_ref):
    i = pl.program_id(0)
    t = h_ref[...] + acc_ref[0] + acc_ref[1]
    u = jnp.maximum(
        jnp.dot(t, wa_ref[...], preferred_element_type=jnp.float32)
        + ba_ref[...], 0.0)
    h3 = jnp.maximum(
        jnp.dot(u, wb_ref[...], preferred_element_type=jnp.float32)
        + bb_ref[...], 0.0)
    seg = lax.broadcasted_iota(jnp.int32, (_G, _BLK), 0)
    bvec = jnp.broadcast_to(batch_ref[0], (_G, _BLK))
    mask = (seg == bvec).astype(jnp.float32)

    @pl.when(i == 0)
    def _():
        sums_ref[...] = jnp.zeros((_G, _D), jnp.float32)
        cnt_ref[...] = jnp.zeros((_G, _D), jnp.float32)

    sums_ref[...] += jnp.dot(mask, h3, preferred_element_type=jnp.float32)
    cnt_ref[...] += jnp.broadcast_to(
        jnp.sum(mask, axis=1, keepdims=True), (_G, _D))

    @pl.when(i == _N // _BLK - 1)
    def _():
        pooled = sums_ref[...] / jnp.maximum(cnt_ref[...], 1.0)
        hid = jnp.maximum(
            jnp.dot(pooled, wh_ref[...], preferred_element_type=jnp.float32)
            + bh_ref[...], 0.0)
        out_ref[...] = (
            jnp.dot(hid, wo_ref[...], preferred_element_type=jnp.float32)
            + bo_ref[...])


def _tc_final(h, acc, wa, ba, wb, bb, batch_r, wh, bh, wo, bo):
    return pl.pallas_call(
        _tc_final_body,
        grid=(_N // _BLK,),
        in_specs=[
            pl.BlockSpec((_BLK, _D), lambda i: (i, 0)),
            pl.BlockSpec((_NC, _BLK, _D), lambda i: (0, i, 0)),
            pl.BlockSpec((_D, _D), lambda i: (0, 0)),
            pl.BlockSpec((1, _D), lambda i: (0, 0)),
            pl.BlockSpec((_D, _D), lambda i: (0, 0)),
            pl.BlockSpec((1, _D), lambda i: (0, 0)),
            pl.BlockSpec((1, 1, _BLK), lambda i: (i, 0, 0)),
            pl.BlockSpec((_D, _D), lambda i: (0, 0)),
            pl.BlockSpec((1, _D), lambda i: (0, 0)),
            pl.BlockSpec((_D, _D), lambda i: (0, 0)),
            pl.BlockSpec((1, _D), lambda i: (0, 0)),
        ],
        out_specs=pl.BlockSpec((_G, _D), lambda i: (0, 0)),
        out_shape=jax.ShapeDtypeStruct((_G, _D), jnp.float32),
        scratch_shapes=[
            pltpu.VMEM((_G, _D), jnp.float32),
            pltpu.VMEM((_G, _D), jnp.float32),
        ],
    )(h, acc, wa, ba, wb, bb, batch_r, wh, bh, wo, bo)


def kernel(x, edge_index, batch, W0, b0, W1a, b1a, W1b, b1b,
           W2a, b2a, W2b, b2b, Wh, bh, Wo, bo):
    pad = _EPAD - _E
    src_r = jnp.concatenate(
        [edge_index[0], jnp.full((pad,), _N, jnp.int32)]).reshape(
            _NW, _CHUNKS, _LN)
    dst_r = jnp.concatenate(
        [edge_index[1], jnp.full((pad,), _N, jnp.int32)]).reshape(
            _NW, _CHUNKS, _LN)
    zeros1 = jnp.zeros((_NACC,), jnp.float32)
    zeros2 = jnp.zeros((_NACC, _D), jnp.float32)
    b0r = b0.reshape(1, _D)
    b1ar, b1br = b1a.reshape(1, _D), b1b.reshape(1, _D)
    b2ar, b2br = b2a.reshape(1, _D), b2b.reshape(1, _D)
    bhr, bor = bh.reshape(1, _D), bo.reshape(1, _D)
    batch_r = batch.reshape(_N // _BLK, 1, _BLK)

    deg_parts = _sc_degree_kernel()(dst_r, zeros1)
    deg = deg_parts[0, :_N] + deg_parts[1, :_N] + 1.0
    invb = jnp.broadcast_to(lax.rsqrt(deg)[:, None], (_N, _D))

    hs, selfterm = _tc_gcn_pre(x, invb, W0, b0r)
    acc0 = _sc_edge_scatter_kernel()(hs, src_r, dst_r, zeros2)
    h1 = _tc_gcn_post(acc0, selfterm, invb)
    acc1 = _sc_edge_scatter_kernel()(h1, src_r, dst_r, zeros2)
    h2 = _tc_gin(h1, acc1, W1a, b1ar, W1b, b1br)
    acc2 = _sc_edge_scatter_kernel()(h2, src_r, dst_r, zeros2)
    return _tc_final(h2, acc2, W2a, b2ar, W2b, b2br, batch_r,
                     Wh, bhr, Wo, bor)
